# baseline (speedup 1.0000x reference)
import functools

import jax
import jax.numpy as jnp
from jax import lax
from jax.experimental import pallas as pl
from jax.experimental.pallas import tpu as pltpu
from jax.experimental.pallas import tpu_sc as plsc

N_NODES = 100000
N_EDGES = 1600000
N_EDGES_H = N_EDGES // 2
HIDDEN = 32
HALF = 16
SUB = 128
CH = 5
CHUNK = CH * SUB
N_CHUNKS = N_EDGES_H // CHUNK
N_TILES = 16
ROWS_PER_TILE = N_NODES // N_TILES
ZROWS = 250

_sc_mesh = plsc.VectorSubcoreMesh(core_axis_name="c", subcore_axis_name="s")


def _make_sc_scatter(col_base: int, with_init: bool):
    def body(refs):
        if with_init:
            (col_hbm, ea_hbm, init_hbm, out_hbm, acc, colv0, colv1, eav0,
             eav1, zb, sc0, sc1, se0, se1) = refs
        else:
            (col_hbm, ea_hbm, out_hbm, acc, colv0, colv1, eav0, eav1, zb,
             sc0, sc1, se0, se1) = refs
        c = lax.axis_index("c")
        s = lax.axis_index("s")
        c16 = c * HALF

        def _load(j, colv, eav, semc, seme):
            for k in range(CH):
                pltpu.async_copy(
                    col_hbm.at[pl.ds(col_base + j * CHUNK + k * SUB, SUB)],
                    colv.at[k], semc,
                )
            pltpu.async_copy(
                ea_hbm.at[pl.ds(j * CHUNK, CHUNK), pl.ds(c16, HALF)], eav, seme
            )

        def _drain(j, colv, eav, semc, seme):
            for k in range(CH):
                pltpu.make_async_copy(
                    col_hbm.at[pl.ds(col_base + j * CHUNK + k * SUB, SUB)],
                    colv.at[k], semc,
                ).wait()
            pltpu.make_async_copy(
                ea_hbm.at[pl.ds(j * CHUNK, CHUNK), pl.ds(c16, HALF)], eav, seme
            ).wait()

        def _scatter(colv, eav):
            for k in range(CH):
                pltpu.sync_copy(
                    eav.at[pl.ds(k * SUB, SUB)], acc.at[colv.at[k]], add=True
                )

        lo = s * N_CHUNKS // N_TILES
        hi = (s + 1) * N_CHUNKS // N_TILES
        n = hi - lo

        _load(lo, colv0, eav0, sc0, se0)

        @pl.when(n > 1)
        def _():
            _load(lo + 1, colv1, eav1, sc1, se1)

        row0 = s * ROWS_PER_TILE
        if with_init:
            pltpu.sync_copy(
                init_hbm.at[c, pl.ds(row0, ROWS_PER_TILE)],
                acc.at[pl.ds(row0, ROWS_PER_TILE)],
            )
        else:
            def _zero_row(i, _):
                zb[i, :] = jnp.zeros((HALF,), jnp.float32)
                return _

            lax.fori_loop(0, ZROWS, _zero_row, None, unroll=4)
            for k in range(ROWS_PER_TILE // ZROWS):
                pltpu.sync_copy(zb, acc.at[pl.ds(row0 + k * ZROWS, ZROWS)])
        plsc.subcore_barrier()

        def _pair(p, _):
            j0 = lo + 2 * p
            _drain(j0, colv0, eav0, sc0, se0)
            _scatter(colv0, eav0)

            @pl.when(j0 + 2 < hi)
            def _():
                _load(j0 + 2, colv0, eav0, sc0, se0)

            _drain(j0 + 1, colv1, eav1, sc1, se1)
            _scatter(colv1, eav1)

            @pl.when(j0 + 3 < hi)
            def _():
                _load(j0 + 3, colv1, eav1, sc1, se1)

            return _

        lax.fori_loop(0, n // 2, _pair, None)

        @pl.when(n % 2 == 1)
        def _():
            j = lo + (n // 2) * 2
            _drain(j, colv0, eav0, sc0, se0)
            _scatter(colv0, eav0)

        plsc.subcore_barrier()

        pltpu.sync_copy(
            acc.at[pl.ds(row0, ROWS_PER_TILE)],
            out_hbm.at[c, pl.ds(row0, ROWS_PER_TILE)],
        )

    def wrapped(*args):
        return body(args)

    return functools.partial(
        pl.kernel,
        out_type=jax.ShapeDtypeStruct((2, N_NODES, HALF), jnp.float32),
        mesh=_sc_mesh,
        scratch_types=[
            pltpu.VMEM_SHARED((N_NODES, HALF), jnp.float32),
            pltpu.VMEM((CH, SUB), jnp.int32),
            pltpu.VMEM((CH, SUB), jnp.int32),
            pltpu.VMEM((CHUNK, HALF), jnp.float32),
            pltpu.VMEM((CHUNK, HALF), jnp.float32),
            pltpu.VMEM((ZROWS, HALF), jnp.float32),
            pltpu.SemaphoreType.DMA,
            pltpu.SemaphoreType.DMA,
            pltpu.SemaphoreType.DMA,
            pltpu.SemaphoreType.DMA,
        ],
        compiler_params=pltpu.CompilerParams(use_tc_tiling_on_sc=False),
    )(wrapped)


_sc_scatter_a = _make_sc_scatter(col_base=0, with_init=False)
_sc_scatter_b = _make_sc_scatter(col_base=N_EDGES_H, with_init=True)


def _mlp_body(x_ref, agg_ref, w1_ref, b1_ref, w2_ref, b2_ref, o_ref):
    x = x_ref[...]
    a = jnp.concatenate([agg_ref[0], agg_ref[1]], axis=1)
    h = (
        jnp.dot(x, w1_ref[:128, :], preferred_element_type=jnp.float32)
        + jnp.dot(a, w1_ref[128:, :], preferred_element_type=jnp.float32)
        + b1_ref[...]
    )
    h = jnp.maximum(h, 0.0)
    o = jnp.dot(h, w2_ref[...], preferred_element_type=jnp.float32) + b2_ref[...]
    o_ref[...] = jnp.maximum(o, 0.0)


_MLP_R = 1000


def _mlp(x, agg2, W1, b1, W2, b2):
    return pl.pallas_call(
        _mlp_body,
        grid=(N_NODES // _MLP_R,),
        in_specs=[
            pl.BlockSpec((_MLP_R, 128), lambda i: (i, 0)),
            pl.BlockSpec((2, _MLP_R, HALF), lambda i: (0, i, 0)),
            pl.BlockSpec((160, 32), lambda i: (0, 0)),
            pl.BlockSpec((1, 32), lambda i: (0, 0)),
            pl.BlockSpec((32, 32), lambda i: (0, 0)),
            pl.BlockSpec((1, 32), lambda i: (0, 0)),
        ],
        out_specs=pl.BlockSpec((_MLP_R, 32), lambda i: (i, 0)),
        out_shape=jax.ShapeDtypeStruct((N_NODES, 32), jnp.float32),
    )(x, agg2, W1, b1, W2, b2)


def kernel(x, edge_index, edge_attr, u, batch, W1, b1, W2, b2):
    col = edge_index[1].astype(jnp.int32)
    ea_a = edge_attr[:N_EDGES_H]
    ea_b = edge_attr[N_EDGES_H:]
    agg_a = _sc_scatter_a(col, ea_a)
    agg2 = _sc_scatter_b(col, ea_b, agg_a)
    return _mlp(x, agg2, W1, b1.reshape(1, 32), W2, b2.reshape(1, 32))


# speedup vs baseline: 1.4011x; 1.4011x over previous
import functools

import jax
import jax.numpy as jnp
from jax import lax
from jax.experimental import pallas as pl
from jax.experimental.pallas import tpu as pltpu
from jax.experimental.pallas import tpu_sc as plsc

N_NODES = 100000
N_EDGES = 1600000
N_EDGES_H = N_EDGES // 2
HIDDEN = 32
HALF = 16
SUB = 128
CH = 5
CHUNK = CH * SUB
N_CHUNKS = N_EDGES // CHUNK
N_TILES = 16
ROWS_PER_TILE = N_NODES // N_TILES
ZROWS = 250

_sc_mesh = plsc.VectorSubcoreMesh(core_axis_name="c", subcore_axis_name="s")


def _make_sc_scatter(col_base: int, with_init: bool):
    def body(refs):
        if with_init:
            (col_hbm, ea_hbm, init_hbm, out_hbm, acc, colv0, colv1, eav0,
             eav1, zb, sc0, sc1, se0, se1) = refs
        else:
            (col_hbm, ea_hbm, out_hbm, acc, colv0, colv1, eav0, eav1, zb,
             sc0, sc1, se0, se1) = refs
        c = lax.axis_index("c")
        s = lax.axis_index("s")
        c16 = c * HALF

        def _load(j, colv, eav, semc, seme):
            for k in range(CH):
                pltpu.async_copy(
                    col_hbm.at[pl.ds(col_base + j * CHUNK + k * SUB, SUB)],
                    colv.at[k], semc,
                )
            pltpu.async_copy(
                ea_hbm.at[pl.ds(j * CHUNK, CHUNK), pl.ds(c16, HALF)], eav, seme
            )

        def _drain(j, colv, eav, semc, seme):
            for k in range(CH):
                pltpu.make_async_copy(
                    col_hbm.at[pl.ds(col_base + j * CHUNK + k * SUB, SUB)],
                    colv.at[k], semc,
                ).wait()
            pltpu.make_async_copy(
                ea_hbm.at[pl.ds(j * CHUNK, CHUNK), pl.ds(c16, HALF)], eav, seme
            ).wait()

        def _scatter(colv, eav):
            for k in range(CH):
                pltpu.sync_copy(
                    eav.at[pl.ds(k * SUB, SUB)], acc.at[colv.at[k]], add=True
                )

        lo = s * N_CHUNKS // N_TILES
        hi = (s + 1) * N_CHUNKS // N_TILES
        n = hi - lo

        _load(lo, colv0, eav0, sc0, se0)

        @pl.when(n > 1)
        def _():
            _load(lo + 1, colv1, eav1, sc1, se1)

        row0 = s * ROWS_PER_TILE
        if with_init:
            pltpu.sync_copy(
                init_hbm.at[c, pl.ds(row0, ROWS_PER_TILE)],
                acc.at[pl.ds(row0, ROWS_PER_TILE)],
            )
        else:
            def _zero_row(i, _):
                zb[i, :] = jnp.zeros((HALF,), jnp.float32)
                return _

            lax.fori_loop(0, ZROWS, _zero_row, None, unroll=4)
            for k in range(ROWS_PER_TILE // ZROWS):
                pltpu.sync_copy(zb, acc.at[pl.ds(row0 + k * ZROWS, ZROWS)])
        plsc.subcore_barrier()

        def _pair(p, _):
            j0 = lo + 2 * p
            _drain(j0, colv0, eav0, sc0, se0)
            _scatter(colv0, eav0)

            @pl.when(j0 + 2 < hi)
            def _():
                _load(j0 + 2, colv0, eav0, sc0, se0)

            _drain(j0 + 1, colv1, eav1, sc1, se1)
            _scatter(colv1, eav1)

            @pl.when(j0 + 3 < hi)
            def _():
                _load(j0 + 3, colv1, eav1, sc1, se1)

            return _

        lax.fori_loop(0, n // 2, _pair, None)

        @pl.when(n % 2 == 1)
        def _():
            j = lo + (n // 2) * 2
            _drain(j, colv0, eav0, sc0, se0)
            _scatter(colv0, eav0)

        plsc.subcore_barrier()

        pltpu.sync_copy(
            acc.at[pl.ds(row0, ROWS_PER_TILE)],
            out_hbm.at[c, pl.ds(row0, ROWS_PER_TILE)],
        )

    def wrapped(*args):
        return body(args)

    return functools.partial(
        pl.kernel,
        out_type=jax.ShapeDtypeStruct((2, N_NODES, HALF), jnp.float32),
        mesh=_sc_mesh,
        scratch_types=[
            pltpu.VMEM_SHARED((N_NODES, HALF), jnp.float32),
            pltpu.VMEM((CH, SUB), jnp.int32),
            pltpu.VMEM((CH, SUB), jnp.int32),
            pltpu.VMEM((CHUNK, HALF), jnp.float32),
            pltpu.VMEM((CHUNK, HALF), jnp.float32),
            pltpu.VMEM((ZROWS, HALF), jnp.float32),
            pltpu.SemaphoreType.DMA,
            pltpu.SemaphoreType.DMA,
            pltpu.SemaphoreType.DMA,
            pltpu.SemaphoreType.DMA,
        ],
        compiler_params=pltpu.CompilerParams(use_tc_tiling_on_sc=False),
    )(wrapped)


_sc_scatter = _make_sc_scatter(col_base=0, with_init=False)


def _mlp_body(x_ref, agg_ref, w1_ref, b1_ref, w2_ref, b2_ref, o_ref):
    x = x_ref[...]
    a = jnp.concatenate([agg_ref[0], agg_ref[1]], axis=1)
    h = (
        jnp.dot(x, w1_ref[:128, :], preferred_element_type=jnp.float32)
        + jnp.dot(a, w1_ref[128:, :], preferred_element_type=jnp.float32)
        + b1_ref[...]
    )
    h = jnp.maximum(h, 0.0)
    o = jnp.dot(h, w2_ref[...], preferred_element_type=jnp.float32) + b2_ref[...]
    o_ref[...] = jnp.maximum(o, 0.0)


_MLP_R = 1000


def _mlp(x, agg2, W1, b1, W2, b2):
    return pl.pallas_call(
        _mlp_body,
        grid=(N_NODES // _MLP_R,),
        in_specs=[
            pl.BlockSpec((_MLP_R, 128), lambda i: (i, 0)),
            pl.BlockSpec((2, _MLP_R, HALF), lambda i: (0, i, 0)),
            pl.BlockSpec((160, 32), lambda i: (0, 0)),
            pl.BlockSpec((1, 32), lambda i: (0, 0)),
            pl.BlockSpec((32, 32), lambda i: (0, 0)),
            pl.BlockSpec((1, 32), lambda i: (0, 0)),
        ],
        out_specs=pl.BlockSpec((_MLP_R, 32), lambda i: (i, 0)),
        out_shape=jax.ShapeDtypeStruct((N_NODES, 32), jnp.float32),
    )(x, agg2, W1, b1, W2, b2)


def kernel(x, edge_index, edge_attr, u, batch, W1, b1, W2, b2):
    col = edge_index[1].astype(jnp.int32)
    agg2 = _sc_scatter(col, edge_attr)
    return _mlp(x, agg2, W1, b1.reshape(1, 32), W2, b2.reshape(1, 32))


# speedup vs baseline: 1.4028x; 1.0012x over previous
import functools

import jax
import jax.numpy as jnp
from jax import lax
from jax.experimental import pallas as pl
from jax.experimental.pallas import tpu as pltpu
from jax.experimental.pallas import tpu_sc as plsc

N_NODES = 100000
N_EDGES = 1600000
N_EDGES_H = N_EDGES // 2
HIDDEN = 32
HALF = 16
SUB = 128
CH = 5
CHUNK = CH * SUB
N_CHUNKS = N_EDGES // CHUNK
N_TILES = 16
ROWS_PER_TILE = N_NODES // N_TILES
ZROWS = 250

_sc_mesh = plsc.VectorSubcoreMesh(core_axis_name="c", subcore_axis_name="s")


def _make_sc_scatter(col_base: int, with_init: bool):
    def body(refs):
        if with_init:
            (col_hbm, ea_hbm, init_hbm, out_hbm, acc, colv0, colv1, eav0,
             eav1, zb, sc0, sc1, se0, se1) = refs
        else:
            (col_hbm, ea_hbm, out_hbm, acc, colv0, colv1, eav0, eav1, zb,
             sc0, sc1, se0, se1) = refs
        c = lax.axis_index("c")
        s = lax.axis_index("s")
        c16 = c * HALF

        def _load(j, colv, eav, semc, seme):
            pltpu.async_copy(
                col_hbm.at[pl.ds(col_base // SUB + j * CH, CH)], colv, semc
            )
            pltpu.async_copy(
                ea_hbm.at[pl.ds(j * CHUNK, CHUNK), pl.ds(c16, HALF)], eav, seme
            )

        def _drain(j, colv, eav, semc, seme):
            pltpu.make_async_copy(
                col_hbm.at[pl.ds(col_base // SUB + j * CH, CH)], colv, semc
            ).wait()
            pltpu.make_async_copy(
                ea_hbm.at[pl.ds(j * CHUNK, CHUNK), pl.ds(c16, HALF)], eav, seme
            ).wait()

        def _scatter(colv, eav):
            for k in range(CH):
                pltpu.sync_copy(
                    eav.at[pl.ds(k * SUB, SUB)], acc.at[colv.at[k]], add=True
                )

        lo = s * N_CHUNKS // N_TILES
        hi = (s + 1) * N_CHUNKS // N_TILES
        n = hi - lo

        _load(lo, colv0, eav0, sc0, se0)

        @pl.when(n > 1)
        def _():
            _load(lo + 1, colv1, eav1, sc1, se1)

        row0 = s * ROWS_PER_TILE
        if with_init:
            pltpu.sync_copy(
                init_hbm.at[c, pl.ds(row0, ROWS_PER_TILE)],
                acc.at[pl.ds(row0, ROWS_PER_TILE)],
            )
        else:
            def _zero_row(i, _):
                zb[i, :] = jnp.zeros((HALF,), jnp.float32)
                return _

            lax.fori_loop(0, ZROWS, _zero_row, None, unroll=4)
            for k in range(ROWS_PER_TILE // ZROWS):
                pltpu.sync_copy(zb, acc.at[pl.ds(row0 + k * ZROWS, ZROWS)])
        plsc.subcore_barrier()

        def _pair(p, _):
            j0 = lo + 2 * p
            _drain(j0, colv0, eav0, sc0, se0)
            _scatter(colv0, eav0)

            @pl.when(j0 + 2 < hi)
            def _():
                _load(j0 + 2, colv0, eav0, sc0, se0)

            _drain(j0 + 1, colv1, eav1, sc1, se1)
            _scatter(colv1, eav1)

            @pl.when(j0 + 3 < hi)
            def _():
                _load(j0 + 3, colv1, eav1, sc1, se1)

            return _

        lax.fori_loop(0, n // 2, _pair, None)

        @pl.when(n % 2 == 1)
        def _():
            j = lo + (n // 2) * 2
            _drain(j, colv0, eav0, sc0, se0)
            _scatter(colv0, eav0)

        plsc.subcore_barrier()

        pltpu.sync_copy(
            acc.at[pl.ds(row0, ROWS_PER_TILE)],
            out_hbm.at[c, pl.ds(row0, ROWS_PER_TILE)],
        )

    def wrapped(*args):
        return body(args)

    return functools.partial(
        pl.kernel,
        out_type=jax.ShapeDtypeStruct((2, N_NODES, HALF), jnp.float32),
        mesh=_sc_mesh,
        scratch_types=[
            pltpu.VMEM_SHARED((N_NODES, HALF), jnp.float32),
            pltpu.VMEM((CH, SUB), jnp.int32),
            pltpu.VMEM((CH, SUB), jnp.int32),
            pltpu.VMEM((CHUNK, HALF), jnp.float32),
            pltpu.VMEM((CHUNK, HALF), jnp.float32),
            pltpu.VMEM((ZROWS, HALF), jnp.float32),
            pltpu.SemaphoreType.DMA,
            pltpu.SemaphoreType.DMA,
            pltpu.SemaphoreType.DMA,
            pltpu.SemaphoreType.DMA,
        ],
        compiler_params=pltpu.CompilerParams(use_tc_tiling_on_sc=False),
    )(wrapped)


_sc_scatter = _make_sc_scatter(col_base=0, with_init=False)


def _mlp_body(x_ref, agg_ref, w1_ref, b1_ref, w2_ref, b2_ref, o_ref):
    x = x_ref[...]
    a = jnp.concatenate([agg_ref[0], agg_ref[1]], axis=1)
    h = (
        jnp.dot(x, w1_ref[:128, :], preferred_element_type=jnp.float32)
        + jnp.dot(a, w1_ref[128:, :], preferred_element_type=jnp.float32)
        + b1_ref[...]
    )
    h = jnp.maximum(h, 0.0)
    o = jnp.dot(h, w2_ref[...], preferred_element_type=jnp.float32) + b2_ref[...]
    o_ref[...] = jnp.maximum(o, 0.0)


_MLP_R = 1000


def _mlp(x, agg2, W1, b1, W2, b2):
    return pl.pallas_call(
        _mlp_body,
        grid=(N_NODES // _MLP_R,),
        in_specs=[
            pl.BlockSpec((_MLP_R, 128), lambda i: (i, 0)),
            pl.BlockSpec((2, _MLP_R, HALF), lambda i: (0, i, 0)),
            pl.BlockSpec((160, 32), lambda i: (0, 0)),
            pl.BlockSpec((1, 32), lambda i: (0, 0)),
            pl.BlockSpec((32, 32), lambda i: (0, 0)),
            pl.BlockSpec((1, 32), lambda i: (0, 0)),
        ],
        out_specs=pl.BlockSpec((_MLP_R, 32), lambda i: (i, 0)),
        out_shape=jax.ShapeDtypeStruct((N_NODES, 32), jnp.float32),
    )(x, agg2, W1, b1, W2, b2)


def kernel(x, edge_index, edge_attr, u, batch, W1, b1, W2, b2):
    col = edge_index[1].astype(jnp.int32).reshape(N_EDGES // SUB, SUB)
    agg2 = _sc_scatter(col, edge_attr)
    return _mlp(x, agg2, W1, b1.reshape(1, 32), W2, b2.reshape(1, 32))
